# baseline (speedup 1.0000x reference)
import functools

import jax
import jax.numpy as jnp
from jax.experimental import pallas as pl
from jax.experimental.pallas import tpu as pltpu


def _rollout_kernel(x_ref, h0_ref, w1_ref, b1_ref, wih_ref, bstore_ref,
                    wcat_a_ref, wcat_b_ref, bhh_ref, w2_ref, b2_ref,
                    q_ref, hout_ref, h_scr, a_scr, gi_scr, *, chunk):
    t = pl.program_id(0)
    H = h_scr.shape[-1]

    @pl.when(t == 0)
    def _():
        h_scr[...] = h0_ref[...]

    w1 = w1_ref[...]
    wih = wih_ref[...]
    b1 = b1_ref[...]

    bstore = bstore_ref[...]
    B = x_ref.shape[1]
    xb = x_ref[...].reshape(chunk * B, x_ref.shape[2]).astype(jnp.bfloat16)
    a = jnp.dot(xb, w1, preferred_element_type=jnp.float32) + b1
    a_scr[...] = jnp.maximum(a, 0.0).astype(jnp.bfloat16)
    gi_scr[...] = (jnp.dot(a_scr[...], wih, preferred_element_type=jnp.float32)
                   + bstore)

    wcat_a = wcat_a_ref[...]
    wcat_b = wcat_b_ref[...]
    w2 = w2_ref[...]
    bhh_n = bhh_ref[:, 2 * H:3 * H]
    b2 = b2_ref[...]
    A = q_ref.shape[-1]
    NSPLIT = 2
    HB = B // NSPLIT

    halves = [h_scr[s * HB:(s + 1) * HB, :] for s in range(NSPLIT)]
    for i in range(chunk):
        for s in range(NSPLIT):
            h = halves[s]
            hb = h.astype(jnp.bfloat16)
            c_a = jnp.dot(hb, wcat_a, preferred_element_type=jnp.float32)
            c_b = jnp.dot(hb, wcat_b, preferred_element_type=jnp.float32)
            if i > 0:
                q_ref[i - 1, s * HB:(s + 1) * HB, :] = c_b[:, H:H + A] + b2
            lo = i * B + s * HB
            hi = lo + HB
            r = jax.nn.sigmoid(gi_scr[lo:hi, 0:H] + c_a[:, 0:H])
            z = jax.nn.sigmoid(gi_scr[lo:hi, H:2 * H] + c_a[:, H:2 * H])
            n = jnp.tanh(gi_scr[lo:hi, 2 * H:3 * H]
                         + r * (c_b[:, 0:H] + bhh_n))
            halves[s] = n + z * (h - n)

    h_final = jnp.concatenate(halves, axis=0)
    q_ref[chunk - 1] = jnp.dot(h_final.astype(jnp.bfloat16), w2,
                               preferred_element_type=jnp.float32) + b2
    h_scr[...] = h_final
    hout_ref[...] = h_final


def _rollout(x_seq, h0, w1t, b1, w_ih_t, b_ih, w_hh_t, b_hh, w2t, b2,
             *, chunk):
    T, B, in_dim = x_seq.shape
    H = h0.shape[1]
    A = w2t.shape[1]
    assert T % chunk == 0

    bf = jnp.bfloat16
    w1b, wihb, whhb, w2b = (w.astype(bf) for w in (w1t, w_ih_t, w_hh_t, w2t))
    w2pad = jnp.zeros((H, 128), bf).at[:, :A].set(w2b)
    wcat_a = whhb[:, 0:2 * H]
    wcat_b = jnp.concatenate([whhb[:, 2 * H:3 * H], w2pad], axis=1)
    bsum = b_ih + b_hh
    bstore = jnp.concatenate([bsum[:, 0:2 * H], b_ih[:, 2 * H:3 * H]], axis=1)

    def wspec(arr):
        return pl.BlockSpec(arr.shape, lambda t: (0,) * arr.ndim)

    grid = (T // chunk,)
    body = functools.partial(_rollout_kernel, chunk=chunk)

    q_seq, h_final = pl.pallas_call(
        body,
        out_shape=(
            jax.ShapeDtypeStruct((T, B, A), jnp.float32),
            jax.ShapeDtypeStruct((B, H), jnp.float32),
        ),
        grid_spec=pltpu.PrefetchScalarGridSpec(
            num_scalar_prefetch=0,
            grid=grid,
            in_specs=[
                pl.BlockSpec((chunk, B, in_dim), lambda t: (t, 0, 0)),
                pl.BlockSpec((B, H), lambda t: (0, 0)),
                wspec(w1b), wspec(b1),
                wspec(wihb), wspec(bstore),
                wspec(wcat_a), wspec(wcat_b), wspec(b_hh),
                wspec(w2b), wspec(b2),
            ],
            out_specs=(
                pl.BlockSpec((chunk, B, A), lambda t: (t, 0, 0)),
                pl.BlockSpec((B, H), lambda t: (0, 0)),
            ),
            scratch_shapes=[
                pltpu.VMEM((B, H), jnp.float32),
                pltpu.VMEM((chunk * B, H), jnp.bfloat16),
                pltpu.VMEM((chunk * B, 3 * H), jnp.float32),
            ],
        ),
        compiler_params=pltpu.CompilerParams(
            dimension_semantics=("arbitrary",)),
    )(
        x_seq, h0,
        w1b, b1, wihb, bstore, wcat_a, wcat_b, b_hh, w2b, b2,
    )
    return q_seq, h_final


def kernel(x_seq, h0, w1t, b1, w_ih_t, b_ih, w_hh_t, b_hh, w2t, b2):
    return _rollout(x_seq, h0, w1t, b1, w_ih_t, b_ih, w_hh_t, b_hh, w2t, b2,
                    chunk=8)


# speedup vs baseline: 1.0207x; 1.0207x over previous
import functools

import jax
import jax.numpy as jnp
from jax.experimental import pallas as pl
from jax.experimental.pallas import tpu as pltpu


def _rollout_kernel(x_ref, h0_ref, w1_ref, b1_ref, wih_ref, bstore_ref,
                    wcat_a_ref, wcat_b_ref, bhh_ref, w2_ref, b2_ref,
                    q_ref, hout_ref, h_scr, a_scr, gi_scr, *, chunk):
    t = pl.program_id(0)
    H = h_scr.shape[-1]

    @pl.when(t == 0)
    def _():
        h_scr[...] = h0_ref[...]

    w1 = w1_ref[...]
    wih = wih_ref[...]
    b1 = b1_ref[...]

    bstore = bstore_ref[...]
    B = x_ref.shape[1]
    xb = x_ref[...].reshape(chunk * B, x_ref.shape[2]).astype(jnp.bfloat16)
    a = jnp.dot(xb, w1, preferred_element_type=jnp.float32) + b1
    a_scr[...] = jnp.maximum(a, 0.0).astype(jnp.bfloat16)
    gi_scr[...] = (jnp.dot(a_scr[...], wih, preferred_element_type=jnp.float32)
                   + bstore)

    wcat_a = wcat_a_ref[...]
    wcat_b = wcat_b_ref[...]
    w2 = w2_ref[...]
    bhh_n = bhh_ref[:, 2 * H:3 * H]
    b2 = b2_ref[...]
    A = q_ref.shape[-1]
    NSPLIT = 2
    HB = B // NSPLIT

    halves = [h_scr[s * HB:(s + 1) * HB, :] for s in range(NSPLIT)]
    for i in range(chunk):
        for s in range(NSPLIT):
            h = halves[s]
            hb = h.astype(jnp.bfloat16)
            c_a = jnp.dot(hb, wcat_a, preferred_element_type=jnp.float32)
            c_b = jnp.dot(hb, wcat_b, preferred_element_type=jnp.float32)
            if i > 0:
                q_ref[i - 1, s * HB:(s + 1) * HB, :] = c_b[:, H:H + A] + b2
            lo = i * B + s * HB
            hi = lo + HB
            r = jax.nn.sigmoid(gi_scr[lo:hi, 0:H] + c_a[:, 0:H])
            z = jax.nn.sigmoid(gi_scr[lo:hi, H:2 * H] + c_a[:, H:2 * H])
            n = jnp.tanh(gi_scr[lo:hi, 2 * H:3 * H]
                         + r * (c_b[:, 0:H] + bhh_n))
            halves[s] = n + z * (h - n)

    h_final = jnp.concatenate(halves, axis=0)
    q_ref[chunk - 1] = jnp.dot(h_final.astype(jnp.bfloat16), w2,
                               preferred_element_type=jnp.float32) + b2
    h_scr[...] = h_final
    hout_ref[...] = h_final


def _rollout(x_seq, h0, w1t, b1, w_ih_t, b_ih, w_hh_t, b_hh, w2t, b2,
             *, chunk):
    T, B, in_dim = x_seq.shape
    H = h0.shape[1]
    A = w2t.shape[1]
    assert T % chunk == 0

    bf = jnp.bfloat16
    w1b, wihb, whhb, w2b = (w.astype(bf) for w in (w1t, w_ih_t, w_hh_t, w2t))
    w2pad = jnp.zeros((H, 128), bf).at[:, :A].set(w2b)
    wcat_a = whhb[:, 0:2 * H]
    wcat_b = jnp.concatenate([whhb[:, 2 * H:3 * H], w2pad], axis=1)
    bsum = b_ih + b_hh
    bstore = jnp.concatenate([bsum[:, 0:2 * H], b_ih[:, 2 * H:3 * H]], axis=1)

    def wspec(arr):
        return pl.BlockSpec(arr.shape, lambda t: (0,) * arr.ndim)

    grid = (T // chunk,)
    body = functools.partial(_rollout_kernel, chunk=chunk)

    q_seq, h_final = pl.pallas_call(
        body,
        out_shape=(
            jax.ShapeDtypeStruct((T, B, A), jnp.float32),
            jax.ShapeDtypeStruct((B, H), jnp.float32),
        ),
        grid_spec=pltpu.PrefetchScalarGridSpec(
            num_scalar_prefetch=0,
            grid=grid,
            in_specs=[
                pl.BlockSpec((chunk, B, in_dim), lambda t: (t, 0, 0)),
                pl.BlockSpec((B, H), lambda t: (0, 0)),
                wspec(w1b), wspec(b1),
                wspec(wihb), wspec(bstore),
                wspec(wcat_a), wspec(wcat_b), wspec(b_hh),
                wspec(w2b), wspec(b2),
            ],
            out_specs=(
                pl.BlockSpec((chunk, B, A), lambda t: (t, 0, 0)),
                pl.BlockSpec((B, H), lambda t: (0, 0)),
            ),
            scratch_shapes=[
                pltpu.VMEM((B, H), jnp.float32),
                pltpu.VMEM((chunk * B, H), jnp.bfloat16),
                pltpu.VMEM((chunk * B, 3 * H), jnp.float32),
            ],
        ),
        compiler_params=pltpu.CompilerParams(
            dimension_semantics=("arbitrary",)),
    )(
        x_seq, h0,
        w1b, b1, wihb, bstore, wcat_a, wcat_b, b_hh, w2b, b2,
    )
    return q_seq, h_final


def kernel(x_seq, h0, w1t, b1, w_ih_t, b_ih, w_hh_t, b_hh, w2t, b2):
    return _rollout(x_seq, h0, w1t, b1, w_ih_t, b_ih, w_hh_t, b_hh, w2t, b2,
                    chunk=16)
